# Initial kernel scaffold; baseline (speedup 1.0000x reference)
import functools

import jax
import jax.numpy as jnp
from jax import lax
from jax.experimental import pallas as pl
from jax.experimental.pallas import tpu as pltpu
from jax.experimental.pallas import tpu_sc as plsc

B, R, NUM_NUMERICAL = 4096, 20, 16
N = B * R
NC, NS = 2, 16
NW = NC * NS
ROWS_PER_W = N // NW
CHUNK = 128
NCH = ROWS_PER_W // CHUNK

DIMS_LIST = (16, 32, 32, 16, 16, 16)

TN = 2048
GRID = N // TN


def _sc_gather_body(idx0, idx1, idx2, idx3, idx4, idx5,
                    tab0, tab1, tab2, tab3, tab4, tab5,
                    out0, out1, out2, out3, out4, out5,
                    idx_v, rows32, rows16, sem, sem_out):
    wid = lax.axis_index("s") * NC + lax.axis_index("c")
    base = wid * ROWS_PER_W
    idxs = (idx0, idx1, idx2, idx3, idx4, idx5)
    tabs = (tab0, tab1, tab2, tab3, tab4, tab5)
    outs = (out0, out1, out2, out3, out4, out5)
    order = (0, 1, 3, 2, 4, 5)
    prev_by_buf = {16: None, 32: None}
    for t in order:
        d = DIMS_LIST[t]
        rows_v = rows32 if d == 32 else rows16
        pltpu.sync_copy(idxs[t].at[wid], idx_v)
        if prev_by_buf[d] is not None:
            prev_by_buf[d].wait()

        def fire(j, _, tab=tabs[t], rv=rows_v):
            pltpu.async_copy(tab.at[idx_v.at[j]],
                             rv.at[pl.ds(j * CHUNK, CHUNK)], sem)
            return 0

        def drain(j, _, tab=tabs[t], rv=rows_v):
            pltpu.make_async_copy(tab.at[idx_v.at[0]],
                                  rv.at[pl.ds(0, CHUNK)], sem).wait()
            return 0

        lax.fori_loop(0, NCH, fire, 0)
        lax.fori_loop(0, NCH, drain, 0)
        prev_by_buf[d] = pltpu.async_copy(
            rows_v, outs[t].at[pl.ds(base, ROWS_PER_W)], sem_out)
    for cp in prev_by_buf.values():
        if cp is not None:
            cp.wait()


@functools.partial(
    pl.kernel,
    out_type=tuple(jax.ShapeDtypeStruct((N, d), jnp.float32)
                   for d in DIMS_LIST),
    mesh=plsc.VectorSubcoreMesh(core_axis_name="c", subcore_axis_name="s",
                                num_cores=NC, num_subcores=NS),
    scratch_types=[
        pltpu.VMEM((NCH, CHUNK), jnp.int32),
        pltpu.VMEM((ROWS_PER_W, 32), jnp.float32),
        pltpu.VMEM((ROWS_PER_W, 16), jnp.float32),
        pltpu.SemaphoreType.DMA,
        pltpu.SemaphoreType.DMA,
    ],
)
def _sc_gather(*args):
    _sc_gather_body(*args)


def _mlp_body(p0, p1, p2, p3, p4, p5, xn, w1, b1, w2t, b2, out):
    x = jnp.concatenate(
        [p0[...], p1[...], p2[...], p3[...], p4[...], p5[...], xn[...]],
        axis=1)
    h = jnp.maximum(
        jnp.dot(x, w1[...], preferred_element_type=jnp.float32,
                precision=lax.Precision.HIGHEST) + b1[...], 0.0)
    out[...] = lax.dot_general(
        h, w2t[...], (((1,), (1,)), ((), ())),
        preferred_element_type=jnp.float32,
        precision=lax.Precision.HIGHEST) + b2[...]


def _mlp(pieces, xn, w1, b1, w2t, b2):
    in_specs = [pl.BlockSpec((TN, d), lambda i: (i, 0)) for d in DIMS_LIST]
    in_specs.append(pl.BlockSpec((TN, NUM_NUMERICAL), lambda i: (i, 0)))
    in_specs += [
        pl.BlockSpec(w1.shape, lambda i: (0, 0)),
        pl.BlockSpec(b1.shape, lambda i: (0, 0)),
        pl.BlockSpec(w2t.shape, lambda i: (0, 0)),
        pl.BlockSpec(b2.shape, lambda i: (0, 0)),
    ]
    return pl.pallas_call(
        _mlp_body,
        grid=(GRID,),
        in_specs=in_specs,
        out_specs=pl.BlockSpec((TN, 1), lambda i: (i, 0)),
        out_shape=jax.ShapeDtypeStruct((N, 1), jnp.float32),
    )(*pieces, xn, w1, b1, w2t, b2)


def kernel(x_cat_going, x_cat_horse_id, x_cat_jockey_id, x_cat_race_class,
           x_cat_track_id, x_cat_trainer_id, x_num,
           table_going, table_horse_id, table_jockey_id, table_race_class,
           table_track_id, table_trainer_id, W1, b1, W2, b2):
    idxs = [jnp.reshape(x, (NW, NCH, CHUNK)) for x in (
        x_cat_going, x_cat_horse_id, x_cat_jockey_id, x_cat_race_class,
        x_cat_track_id, x_cat_trainer_id)]
    tabs = (table_going, table_horse_id, table_jockey_id, table_race_class,
            table_track_id, table_trainer_id)
    pieces = _sc_gather(*idxs, *tabs)
    logits = _mlp(pieces, jnp.reshape(x_num, (N, NUM_NUMERICAL)),
                  W1, jnp.reshape(b1, (1, 64)), jnp.reshape(W2, (1, 64)),
                  jnp.reshape(b2, (1, 1)))
    return jnp.reshape(logits, (B, R))


# speedup vs baseline: 4.5612x; 4.5612x over previous
import functools

import jax
import jax.numpy as jnp
from jax import lax
from jax.experimental import pallas as pl
from jax.experimental.pallas import tpu as pltpu
from jax.experimental.pallas import tpu_sc as plsc

B, R, NUM_NUMERICAL = 4096, 20, 16
N = B * R
NC, NS = 2, 16
NW = NC * NS
ROWS_PER_W = N // NW
CHUNK = 128
NCH = ROWS_PER_W // CHUNK
NBUF = 4

NTAB = 6
DIMS_LIST = (16, 32, 32, 16, 16, 16)
COL0 = (0, 16, 48, 80, 96, 112)

TB = 128
TN = TB * R
GRID = B // TB


def _sc_body(idx_hbm, tab0, tab1, tab2, tab3, tab4, tab5, out,
             idx_v, b16_0, b16_1, b16_2, b16_3, b32_0, b32_1, b32_2, b32_3,
             sg0, sg1, sg2, sg3, so0, so1, so2, so3):
    wid = lax.axis_index("s") * NC + lax.axis_index("c")
    base = wid * ROWS_PER_W
    tabs = (tab0, tab1, tab2, tab3, tab4, tab5)
    bufs16 = (b16_0, b16_1, b16_2, b16_3)
    bufs32 = (b32_0, b32_1, b32_2, b32_3)
    sgs = (sg0, sg1, sg2, sg3)
    sos = (so0, so1, so2, so3)
    pltpu.sync_copy(idx_hbm.at[wid], idx_v)
    for t in range(NTAB):
        d = DIMS_LIST[t]
        c0 = COL0[t]
        tab = tabs[t]
        bufs = bufs32 if d == 32 else bufs16

        def iter_body(i, _, tab=tab, d=d, c0=c0, t=t, bufs=bufs):
            for k in range(NBUF):
                j = i * NBUF + k

                @pl.when(j >= NBUF)
                def _(k=k, d=d, c0=c0, bufs=bufs):
                    pltpu.make_async_copy(
                        bufs[k],
                        out.at[pl.ds(base, CHUNK), pl.ds(c0, d)],
                        sos[k]).wait()

                pltpu.async_copy(tab.at[idx_v.at[t, j]], bufs[k], sgs[k])
            for k in range(NBUF):
                j = i * NBUF + k
                pltpu.make_async_copy(tab.at[idx_v.at[t, 0]], bufs[k],
                                      sgs[k]).wait()
                pltpu.async_copy(
                    bufs[k],
                    out.at[pl.ds(base + j * CHUNK, CHUNK), pl.ds(c0, d)],
                    sos[k])
            return 0

        lax.fori_loop(0, NCH // NBUF, iter_body, 0)
        for k in range(NBUF):
            pltpu.make_async_copy(
                bufs[k],
                out.at[pl.ds(base, CHUNK), pl.ds(c0, d)],
                sos[k]).wait()


@functools.partial(
    pl.kernel,
    out_type=jax.ShapeDtypeStruct((N, 128), jnp.float32),
    mesh=plsc.VectorSubcoreMesh(core_axis_name="c", subcore_axis_name="s",
                                num_cores=NC, num_subcores=NS),
    compiler_params=pltpu.CompilerParams(use_tc_tiling_on_sc=False),
    scratch_types=[
        pltpu.VMEM((NTAB, NCH, CHUNK), jnp.int32),
        pltpu.VMEM((CHUNK, 16), jnp.float32),
        pltpu.VMEM((CHUNK, 16), jnp.float32),
        pltpu.VMEM((CHUNK, 16), jnp.float32),
        pltpu.VMEM((CHUNK, 16), jnp.float32),
        pltpu.VMEM((CHUNK, 32), jnp.float32),
        pltpu.VMEM((CHUNK, 32), jnp.float32),
        pltpu.VMEM((CHUNK, 32), jnp.float32),
        pltpu.VMEM((CHUNK, 32), jnp.float32),
        pltpu.SemaphoreType.DMA,
        pltpu.SemaphoreType.DMA,
        pltpu.SemaphoreType.DMA,
        pltpu.SemaphoreType.DMA,
        pltpu.SemaphoreType.DMA,
        pltpu.SemaphoreType.DMA,
        pltpu.SemaphoreType.DMA,
        pltpu.SemaphoreType.DMA,
    ],
)
def _sc_gather(*args):
    _sc_body(*args)


def _mlp_body(emb, xn, w1e, w1n, b1r, w2, b2r, out):
    x_num = xn[...].reshape(TN, NUM_NUMERICAL)
    h = jnp.maximum(
        jnp.dot(emb[...], w1e[...], preferred_element_type=jnp.float32,
                precision=lax.Precision.HIGHEST)
        + jnp.dot(x_num, w1n[...], preferred_element_type=jnp.float32,
                  precision=lax.Precision.HIGHEST)
        + b1r[...], 0.0)
    logits = jnp.dot(h, w2[...], preferred_element_type=jnp.float32,
                     precision=lax.Precision.HIGHEST) + b2r[...]
    out[...] = logits.reshape(TB, R)


def _mlp(emb, x_num, W1, b1, W2, b2):
    return pl.pallas_call(
        _mlp_body,
        grid=(GRID,),
        in_specs=[
            pl.BlockSpec((TN, 128), lambda i: (i, 0)),
            pl.BlockSpec((TB, R, NUM_NUMERICAL), lambda i: (i, 0, 0)),
            pl.BlockSpec((128, 64), lambda i: (0, 0)),
            pl.BlockSpec((NUM_NUMERICAL, 64), lambda i: (0, 0)),
            pl.BlockSpec((1, 64), lambda i: (0, 0)),
            pl.BlockSpec((64, 1), lambda i: (0, 0)),
            pl.BlockSpec((1, 1), lambda i: (0, 0)),
        ],
        out_specs=pl.BlockSpec((TB, R), lambda i: (i, 0)),
        out_shape=jax.ShapeDtypeStruct((B, R), jnp.float32),
    )(emb, x_num, W1[:128], W1[128:], b1.reshape(1, 64), W2,
      b2.reshape(1, 1))


def kernel(x_cat_going, x_cat_horse_id, x_cat_jockey_id, x_cat_race_class,
           x_cat_track_id, x_cat_trainer_id, x_num,
           table_going, table_horse_id, table_jockey_id, table_race_class,
           table_track_id, table_trainer_id, W1, b1, W2, b2):
    idx = jnp.stack([jnp.reshape(x, (NW, NCH, CHUNK)) for x in (
        x_cat_going, x_cat_horse_id, x_cat_jockey_id, x_cat_race_class,
        x_cat_track_id, x_cat_trainer_id)], axis=1)
    emb = _sc_gather(idx, table_going, table_horse_id, table_jockey_id,
                     table_race_class, table_track_id, table_trainer_id)
    return _mlp(emb, x_num, W1, b1, W2, b2)


# speedup vs baseline: 5.0303x; 1.1028x over previous
import functools

import jax
import jax.numpy as jnp
from jax import lax
from jax.experimental import pallas as pl
from jax.experimental.pallas import tpu as pltpu
from jax.experimental.pallas import tpu_sc as plsc

B, R, NUM_NUMERICAL = 4096, 20, 16
N = B * R
NC, NS = 2, 16
NW = NC * NS
ROWS_PER_W = N // NW
CHUNK = 128
NCH = ROWS_PER_W // CHUNK
NBUF = 4

NTAB = 6
DIMS_LIST = (16, 32, 32, 16, 16, 16)
COL0 = (0, 16, 48, 80, 96, 112)

TB = 128
TN = TB * R
GRID = B // TB


def _sc_body(idx_hbm, tab0, tab1, tab2, tab3, tab4, tab5, out,
             idx_v, b16_0, b16_1, b16_2, b16_3, b32_0, b32_1, b32_2, b32_3,
             sg0, sg1, sg2, sg3, so0, so1, so2, so3):
    wid = lax.axis_index("s") * NC + lax.axis_index("c")
    base = wid * ROWS_PER_W
    tabs = (tab0, tab1, tab2, tab3, tab4, tab5)
    bufs16 = (b16_0, b16_1, b16_2, b16_3)
    bufs32 = (b32_0, b32_1, b32_2, b32_3)
    sgs = (sg0, sg1, sg2, sg3)
    sos = (so0, so1, so2, so3)
    pltpu.sync_copy(idx_hbm.at[wid], idx_v)
    for t in range(NTAB):
        d = DIMS_LIST[t]
        c0 = COL0[t]
        tab = tabs[t]
        bufs = bufs32 if d == 32 else bufs16

        def iter_body(i, _, tab=tab, d=d, c0=c0, t=t, bufs=bufs):
            for k in range(NBUF):
                j = i * NBUF + k

                @pl.when(j >= NBUF)
                def _(k=k, d=d, c0=c0, bufs=bufs):
                    pltpu.make_async_copy(
                        bufs[k],
                        out.at[pl.ds(base, CHUNK), pl.ds(c0, d)],
                        sos[k]).wait()

                pltpu.async_copy(tab.at[idx_v.at[t, j]], bufs[k], sgs[k])
            for k in range(NBUF):
                j = i * NBUF + k
                pltpu.make_async_copy(tab.at[idx_v.at[t, 0]], bufs[k],
                                      sgs[k]).wait()
                pltpu.async_copy(
                    bufs[k],
                    out.at[pl.ds(base + j * CHUNK, CHUNK), pl.ds(c0, d)],
                    sos[k])
            return 0

        lax.fori_loop(0, NCH // NBUF, iter_body, 0)
        for k in range(NBUF):
            pltpu.make_async_copy(
                bufs[k],
                out.at[pl.ds(base, CHUNK), pl.ds(c0, d)],
                sos[k]).wait()


@functools.partial(
    pl.kernel,
    out_type=jax.ShapeDtypeStruct((N, 128), jnp.float32),
    mesh=plsc.VectorSubcoreMesh(core_axis_name="c", subcore_axis_name="s",
                                num_cores=NC, num_subcores=NS),
    compiler_params=pltpu.CompilerParams(use_tc_tiling_on_sc=False),
    scratch_types=[
        pltpu.VMEM((NTAB, NCH, CHUNK), jnp.int32),
        pltpu.VMEM((CHUNK, 16), jnp.float32),
        pltpu.VMEM((CHUNK, 16), jnp.float32),
        pltpu.VMEM((CHUNK, 16), jnp.float32),
        pltpu.VMEM((CHUNK, 16), jnp.float32),
        pltpu.VMEM((CHUNK, 32), jnp.float32),
        pltpu.VMEM((CHUNK, 32), jnp.float32),
        pltpu.VMEM((CHUNK, 32), jnp.float32),
        pltpu.VMEM((CHUNK, 32), jnp.float32),
        pltpu.SemaphoreType.DMA,
        pltpu.SemaphoreType.DMA,
        pltpu.SemaphoreType.DMA,
        pltpu.SemaphoreType.DMA,
        pltpu.SemaphoreType.DMA,
        pltpu.SemaphoreType.DMA,
        pltpu.SemaphoreType.DMA,
        pltpu.SemaphoreType.DMA,
    ],
)
def _sc_gather(*args):
    _sc_body(*args)


def _mlp_body(emb, xn, w1e, w1n, b1r, w2, b2r, out):
    h = jnp.maximum(
        jnp.dot(emb[...], w1e[...], preferred_element_type=jnp.float32)
        + jnp.dot(xn[...], w1n[...], preferred_element_type=jnp.float32)
        + b1r[...], 0.0)
    logits = jnp.dot(h, w2[...], preferred_element_type=jnp.float32) + b2r[...]
    out[...] = logits.reshape(TB, R)


def _mlp(emb, x_num, W1, b1, W2, b2):
    return pl.pallas_call(
        _mlp_body,
        grid=(GRID,),
        in_specs=[
            pl.BlockSpec((TN, 128), lambda i: (i, 0)),
            pl.BlockSpec((TN, NUM_NUMERICAL), lambda i: (i, 0)),
            pl.BlockSpec((128, 64), lambda i: (0, 0)),
            pl.BlockSpec((NUM_NUMERICAL, 64), lambda i: (0, 0)),
            pl.BlockSpec((1, 64), lambda i: (0, 0)),
            pl.BlockSpec((64, 1), lambda i: (0, 0)),
            pl.BlockSpec((1, 1), lambda i: (0, 0)),
        ],
        out_specs=pl.BlockSpec((TB, R), lambda i: (i, 0)),
        out_shape=jax.ShapeDtypeStruct((B, R), jnp.float32),
    )(emb, jnp.reshape(x_num, (N, NUM_NUMERICAL)), W1[:128], W1[128:],
      b1.reshape(1, 64), W2, b2.reshape(1, 1))


def kernel(x_cat_going, x_cat_horse_id, x_cat_jockey_id, x_cat_race_class,
           x_cat_track_id, x_cat_trainer_id, x_num,
           table_going, table_horse_id, table_jockey_id, table_race_class,
           table_track_id, table_trainer_id, W1, b1, W2, b2):
    idx = jnp.stack([jnp.reshape(x, (NW, NCH, CHUNK)) for x in (
        x_cat_going, x_cat_horse_id, x_cat_jockey_id, x_cat_race_class,
        x_cat_track_id, x_cat_trainer_id)], axis=1)
    emb = _sc_gather(idx, table_going, table_horse_id, table_jockey_id,
                     table_race_class, table_track_id, table_trainer_id)
    return _mlp(emb, x_num, W1, b1, W2, b2)
